# Initial kernel scaffold; baseline (speedup 1.0000x reference)
import jax
import jax.numpy as jnp
from jax.experimental import pallas as pl


def kernel(x, xyz, W, gamma, beta):
    raise NotImplementedError("write your pallas kernel here")


# speedup vs baseline: 102.3538x; 102.3538x over previous
import functools

import jax
import jax.numpy as jnp
from jax import lax
from jax.experimental import pallas as pl
from jax.experimental.pallas import tpu as pltpu
from jax.experimental.pallas import tpu_sc as plsc

_RADIUS = 0.1
_K = 32
_EPS = 1e-5

_TM = 256
_TN = 512
_TZ = 512
_NW = 32
_CQ = 4


def _ballquery_body(qt_ref, s_ref, idx_ref, *, n_pad, n_real, tn, k):
    tm = qt_ref.shape[0]
    r2 = jnp.float32(_RADIUS * _RADIUS)
    qx = qt_ref[:, 0:1]
    qy = qt_ref[:, 1:2]
    qz = qt_ref[:, 2:3]

    row = lax.broadcasted_iota(jnp.int32, (tn, tn), 0)
    col = lax.broadcasted_iota(jnp.int32, (tn, tn), 1)
    tril = (row <= col).astype(jnp.float32)
    slot_ids = lax.broadcasted_iota(jnp.int32, (1, k), 1)

    def jtile(t, carry):
        count, acc = carry
        sx = s_ref[0:1, pl.ds(t * tn, tn)]
        sy = s_ref[1:2, pl.ds(t * tn, tn)]
        sz = s_ref[2:3, pl.ds(t * tn, tn)]
        d2 = (qx - sx) ** 2 + (qy - sy) ** 2 + (qz - sz) ** 2
        within = (d2 < r2).astype(jnp.float32)
        cum = count + jnp.dot(within, tril, preferred_element_type=jnp.float32)

        def sbody(s, a):
            sf = s.astype(jnp.float32)
            cnt = jnp.sum((cum <= sf).astype(jnp.float32), axis=1,
                          keepdims=True)
            return a + cnt * (slot_ids == s).astype(jnp.float32)

        acc = lax.fori_loop(0, k, sbody, acc)
        count = count + jnp.sum(within, axis=1, keepdims=True)
        return count, acc

    init = (jnp.zeros((tm, 1), jnp.float32), jnp.zeros((tm, k), jnp.float32))
    _, acc = lax.fori_loop(0, n_pad // tn, jtile, init)

    valid = acc < jnp.float32(n_real)
    first = acc[:, 0:1]
    firstv = jnp.where(first < jnp.float32(n_real), first, 0.0)
    idx_ref[...] = jnp.where(valid, acc, firstv).astype(jnp.int32)


def _zmat_body(a_ref, b_ref, o_ref):
    o_ref[...] = jnp.dot(a_ref[...], b_ref[...],
                         preferred_element_type=jnp.float32,
                         precision=lax.Precision.HIGHEST)


def _stats_body(h_ref, qt_ref, wx_ref, o_ref, *, n_real, tm):
    i = pl.program_id(0)

    @pl.when(i == 0)
    def _():
        o_ref[...] = jnp.zeros_like(o_ref)

    c = jnp.dot(qt_ref[...], wx_ref[...],
                preferred_element_type=jnp.float32,
                precision=lax.Precision.HIGHEST)
    h = h_ref[...] - c
    rid = i * tm + lax.broadcasted_iota(jnp.int32, (tm, 1), 0)
    m = (rid < n_real).astype(jnp.float32)
    hm = h * m
    o_ref[0:1, :] = o_ref[0:1, :] + jnp.sum(hm, axis=0, keepdims=True)
    o_ref[1:2, :] = o_ref[1:2, :] + jnp.sum(hm * h, axis=0, keepdims=True)


def _bn_body(h_ref, qt_ref, wx_ref, st_ref, g_ref, b_ref, o_ref, *, n_real):
    inv_n = jnp.float32(1.0 / n_real)
    mean = st_ref[0:1, :] * inv_n
    var = st_ref[1:2, :] * inv_n - mean * mean
    inv = lax.rsqrt(var + jnp.float32(_EPS))
    c = jnp.dot(qt_ref[...], wx_ref[...],
                preferred_element_type=jnp.float32,
                precision=lax.Precision.HIGHEST)
    h = h_ref[...] - c
    y = (h - mean) * (inv * g_ref[...]) + b_ref[...]
    o_ref[...] = jnp.maximum(y, 0.0).T


def _gathermax_body(z_hbm, idx_hbm, out_hbm, idx_v, rows_v, out_v, sem,
                    *, per_w, d):
    cid = lax.axis_index("c")
    sid = lax.axis_index("s")
    wid = sid * 2 + cid
    ng = d // 16

    def chunk(i, carry):
        q0 = wid * per_w + i * _CQ
        pltpu.sync_copy(idx_hbm.at[pl.ds(q0 * _K, _CQ * _K)], idx_v)
        pltpu.async_copy(z_hbm.at[idx_v], rows_v, sem).wait()
        for q in range(_CQ):
            for g in range(ng):
                a = rows_v[q * _K, pl.ds(g * 16, 16)]
                for r in range(1, _K):
                    a = jnp.maximum(a, rows_v[q * _K + r, pl.ds(g * 16, 16)])
                out_v[q, pl.ds(g * 16, 16)] = a
        pltpu.sync_copy(out_v, out_hbm.at[pl.ds(q0, _CQ)])
        return carry

    lax.fori_loop(0, per_w // _CQ, chunk, 0)


def kernel(x, xyz, W, gamma, beta):
    b, d_in, n = x.shape
    d_out = W.shape[0]
    m_pad = 10240 if n <= 10240 else ((n + 1023) // 1024) * 1024
    n_pad = m_pad

    xyz2 = xyz[0]
    qt = jnp.full((m_pad, 4), 1e6, jnp.float32)
    qt = qt.at[:n, 0:3].set(xyz2.T)
    src = jnp.full((4, n_pad), -1e6, jnp.float32)
    src = src.at[0:3, :n].set(xyz2)

    idx = pl.pallas_call(
        functools.partial(_ballquery_body, n_pad=n_pad, n_real=n, tn=_TN,
                          k=_K),
        grid=(m_pad // _TM,),
        in_specs=[
            pl.BlockSpec((_TM, 4), lambda i: (i, 0)),
            pl.BlockSpec((4, n_pad), lambda i: (0, 0)),
        ],
        out_specs=pl.BlockSpec((_TM, _K), lambda i: (i, 0)),
        out_shape=jax.ShapeDtypeStruct((m_pad, _K), jnp.int32),
    )(qt, src)

    kdim = d_in + 8
    xa = jnp.zeros((n_pad, kdim), jnp.float32)
    xa = xa.at[:n, :d_in].set(x[0].T)
    xa = xa.at[:n, d_in:d_in + 3].set(xyz2.T / _RADIUS)
    wt = jnp.zeros((kdim, d_out), jnp.float32)
    wt = wt.at[:d_in + 3, :].set(W.T)
    zT = pl.pallas_call(
        _zmat_body,
        grid=(n_pad // _TZ,),
        in_specs=[
            pl.BlockSpec((_TZ, kdim), lambda i: (i, 0)),
            pl.BlockSpec((kdim, d_out), lambda i: (0, 0)),
        ],
        out_specs=pl.BlockSpec((_TZ, d_out), lambda i: (i, 0)),
        out_shape=jax.ShapeDtypeStruct((n_pad, d_out), jnp.float32),
    )(xa, wt)

    per_w = m_pad // _NW
    idx_flat = idx.reshape(m_pad * _K)
    gm = functools.partial(
        pl.kernel,
        mesh=plsc.VectorSubcoreMesh(core_axis_name="c", subcore_axis_name="s"),
        out_type=jax.ShapeDtypeStruct((m_pad, d_out), jnp.float32),
        scratch_types=[
            pltpu.VMEM((_CQ * _K,), jnp.int32),
            pltpu.VMEM((_CQ * _K, d_out), jnp.float32),
            pltpu.VMEM((_CQ, d_out), jnp.float32),
            pltpu.SemaphoreType.DMA,
        ],
    )(functools.partial(_gathermax_body, per_w=per_w, d=d_out))
    hmax = gm(zT, idx_flat)

    wx = jnp.zeros((4, d_out), jnp.float32)
    wx = wx.at[0:3, :].set(W[:, d_in:d_in + 3].T / _RADIUS)
    stats = pl.pallas_call(
        functools.partial(_stats_body, n_real=n, tm=_TM),
        grid=(m_pad // _TM,),
        in_specs=[
            pl.BlockSpec((_TM, d_out), lambda i: (i, 0)),
            pl.BlockSpec((_TM, 4), lambda i: (i, 0)),
            pl.BlockSpec((4, d_out), lambda i: (0, 0)),
        ],
        out_specs=pl.BlockSpec((8, d_out), lambda i: (0, 0)),
        out_shape=jax.ShapeDtypeStruct((8, d_out), jnp.float32),
    )(hmax, qt, wx)

    outT = pl.pallas_call(
        functools.partial(_bn_body, n_real=n),
        grid=(m_pad // _TM,),
        in_specs=[
            pl.BlockSpec((_TM, d_out), lambda i: (i, 0)),
            pl.BlockSpec((_TM, 4), lambda i: (i, 0)),
            pl.BlockSpec((4, d_out), lambda i: (0, 0)),
            pl.BlockSpec((8, d_out), lambda i: (0, 0)),
            pl.BlockSpec((1, d_out), lambda i: (0, 0)),
            pl.BlockSpec((1, d_out), lambda i: (0, 0)),
        ],
        out_specs=pl.BlockSpec((d_out, _TM), lambda i: (0, i)),
        out_shape=jax.ShapeDtypeStruct((d_out, m_pad), jnp.float32),
    )(hmax, qt, wx, stats, gamma[None, :], beta[None, :])

    return outT[:, :n][None]
